# baseline (speedup 1.0000x reference)
import functools

import jax
import jax.numpy as jnp
from jax import lax
from jax.experimental import pallas as pl
from jax.experimental.pallas import tpu as pltpu
from jax.experimental.pallas import tpu_sc as plsc

N = 10000
E = 320000
F_IN = 128
H = 64
L = 3
C = 10
G = 64
BN_EPS = 1e-5

NC = 2
NS = 16
NW = NC * NS
CHUNK = 80
NCHUNK = E // (NW * CHUNK)
N_PAD = 10240
ROWS_PER_TILE = N_PAD // NS
ZROWS = 32
NBUF = 5


HPACK = H // 2
RUNROLL = 8


def _unpack_chunk(gbuf, sbuf, b):
    def urow(r0, carry):
        for rr in range(RUNROLL):
            r = r0 * RUNROLL + rr
            for q in range(2):
                v = gbuf[b, r, pl.ds(16 * q, 16)]
                ab = plsc.bitcast(v, jnp.bfloat16)
                lo, hi = plsc.unpack(ab, format=plsc.PackFormat.INTERLEAVED)
                sbuf[b, r, pl.ds(32 * q, 16)] = lo
                sbuf[b, r, pl.ds(32 * q + 16, 16)] = hi
        return carry

    lax.fori_loop(0, CHUNK // RUNROLL, urow, 0)


def _segsum_body(hp_hbm, ei_hbm, out_hbm,
                 sidx, didx, gbuf, sbuf, zbuf, acc, *sems):
    gsem = sems[:NBUF]
    ssem = sems[NBUF:]
    c = lax.axis_index("c")
    s = lax.axis_index("s")
    wid = c * NS + s

    for r in range(ZROWS):
        for q in range(H // 16):
            zbuf[r, pl.ds(q * 16, 16)] = jnp.zeros((16,), jnp.float32)
    base = s * ROWS_PER_TILE

    def zloop(k, carry):
        pltpu.sync_copy(zbuf, acc.at[pl.ds(base + k * ZROWS, ZROWS)])
        return carry

    lax.fori_loop(0, ROWS_PER_TILE // ZROWS, zloop, 0)

    pltpu.sync_copy(ei_hbm.at[0, wid], sidx)
    pltpu.sync_copy(ei_hbm.at[1, wid], didx)

    plsc.subcore_barrier()

    def wait_gather(b):
        pltpu.make_async_copy(hp_hbm.at[sidx.at[b]],
                              gbuf.at[b], gsem[b]).wait()

    def wait_scatter(b, j):
        pltpu.make_async_copy(sbuf.at[b], acc.at[didx.at[j]],
                              ssem[b]).wait()

    for b in range(NBUF):
        pltpu.async_copy(hp_hbm.at[sidx.at[b]], gbuf.at[b], gsem[b])

    for b in range(NBUF):
        wait_gather(b)
        _unpack_chunk(gbuf, sbuf, b)
        pltpu.async_copy(sbuf.at[b], acc.at[didx.at[b]], ssem[b], add=True)
        pltpu.async_copy(hp_hbm.at[sidx.at[b + NBUF]], gbuf.at[b], gsem[b])

    def group(io, carry):
        jo = io * NBUF
        for b in range(NBUF):
            j = jo + b
            wait_gather(b)
            wait_scatter(b, j - NBUF)
            _unpack_chunk(gbuf, sbuf, b)
            pltpu.async_copy(sbuf.at[b], acc.at[didx.at[j]], ssem[b],
                             add=True)
            pltpu.async_copy(hp_hbm.at[sidx.at[j + NBUF]], gbuf.at[b],
                             gsem[b])
        return carry

    lax.fori_loop(1, NCHUNK // NBUF - 1, group, 0)

    jo = NCHUNK - NBUF
    for b in range(NBUF):
        j = jo + b
        wait_gather(b)
        wait_scatter(b, j - NBUF)
        _unpack_chunk(gbuf, sbuf, b)
        pltpu.async_copy(sbuf.at[b], acc.at[didx.at[j]], ssem[b], add=True)
    for b in range(NBUF):
        wait_scatter(b, jo + b)

    plsc.subcore_barrier()
    pltpu.sync_copy(acc.at[pl.ds(base, ROWS_PER_TILE)],
                    out_hbm.at[c, pl.ds(base, ROWS_PER_TILE)])


_segsum_call = pl.kernel(
    _segsum_body,
    out_type=jax.ShapeDtypeStruct((NC, N_PAD, H), jnp.float32),
    mesh=plsc.VectorSubcoreMesh(core_axis_name="c", subcore_axis_name="s",
                                num_cores=NC, num_subcores=NS),
    scratch_types=[
        pltpu.VMEM((NCHUNK, CHUNK), jnp.int32),
        pltpu.VMEM((NCHUNK, CHUNK), jnp.int32),
        pltpu.VMEM((NBUF, CHUNK, HPACK), jnp.int32),
        pltpu.VMEM((NBUF, CHUNK, H), jnp.float32),
        pltpu.VMEM((ZROWS, H), jnp.float32),
        pltpu.VMEM_SHARED((N_PAD, H), jnp.float32),
    ] + [pltpu.SemaphoreType.DMA] * (2 * NBUF),
    compiler_params=pltpu.CompilerParams(use_tc_tiling_on_sc=False,
                                         needs_layout_passes=False),
    name="gin_segsum_sc",
)


def _enc_body(x_ref, w_ref, b_ref, out_ref):
    out_ref[:, :] = jnp.dot(x_ref[:, :], w_ref[:, :],
                            preferred_element_type=jnp.float32) + b_ref[:, :]


_enc_call = pl.pallas_call(
    _enc_body,
    out_shape=jax.ShapeDtypeStruct((N, H), jnp.float32),
    name="gin_encoder_tc",
)


def _mlp_body(h_ref, parts_ref, w1_ref, b1_ref, w2_ref, b2_ref,
              gam_ref, bet_ref, eps_ref, out_ref):
    a = ((1.0 + eps_ref[0, 0]) * h_ref[:, :]
         + parts_ref[0, :N, :] + parts_ref[1, :N, :])
    t = jnp.dot(a, w1_ref[:, :], preferred_element_type=jnp.float32)
    t = jnp.maximum(t + b1_ref[:, :], 0.0)
    h2 = jnp.dot(t, w2_ref[:, :], preferred_element_type=jnp.float32)
    h2 = h2 + b2_ref[:, :]
    mean = jnp.mean(h2, axis=0, keepdims=True)
    var = jnp.mean((h2 - mean) ** 2, axis=0, keepdims=True)
    hn = (h2 - mean) / jnp.sqrt(var + BN_EPS) * gam_ref[:, :] + bet_ref[:, :]
    out_ref[:, :] = jnp.maximum(hn, 0.0)


_mlp_call = pl.pallas_call(
    _mlp_body,
    out_shape=jax.ShapeDtypeStruct((N, H), jnp.float32),
    name="gin_layer_tc",
)


def _pool_body(h_ref, batch_ref, w_ref, b_ref, out_ref):
    gids = lax.broadcasted_iota(jnp.int32, (G, 1), 0)
    mask = (batch_ref[:, :] == gids).astype(jnp.float32)
    sums = jnp.dot(mask, h_ref[:, :], preferred_element_type=jnp.float32)
    counts = jnp.sum(mask, axis=1, keepdims=True)
    pooled = sums / jnp.maximum(counts, 1.0)
    out_ref[:, :] = jnp.dot(pooled, w_ref[:, :],
                            preferred_element_type=jnp.float32) + b_ref[:, :]


_pool_call = pl.pallas_call(
    _pool_body,
    out_shape=jax.ShapeDtypeStruct((G, C), jnp.float32),
    name="gin_pool_tc",
)


def _pack_table(h):
    hb = h.astype(jnp.bfloat16).reshape(N, 2, 2, 16)
    hp = jnp.stack([hb[:, :, 0, :], hb[:, :, 1, :]], axis=-1)
    return jax.lax.bitcast_convert_type(hp, jnp.int32).reshape(N, HPACK)


def kernel(x, edge_index, batch, enc_W, enc_b, eps, W1, b1, W2, b2,
           gamma, beta, lin_W, lin_b):
    ei = edge_index.reshape(2, NW, NCHUNK, CHUNK)
    h = _enc_call(x, enc_W, enc_b.reshape(1, H))
    for i in range(L):
        parts = _segsum_call(_pack_table(h), ei)
        h = _mlp_call(h, parts, W1[i], b1[i].reshape(1, H),
                      W2[i], b2[i].reshape(1, H), gamma[i].reshape(1, H),
                      beta[i].reshape(1, H), eps[i].reshape(1, 1))
    return _pool_call(h, batch.reshape(1, N), lin_W, lin_b.reshape(1, C))


# speedup vs baseline: 1.7020x; 1.7020x over previous
import functools

import jax
import jax.numpy as jnp
from jax import lax
from jax.experimental import pallas as pl
from jax.experimental.pallas import tpu as pltpu
from jax.experimental.pallas import tpu_sc as plsc

N = 10000
E = 320000
F_IN = 128
H = 64
L = 3
C = 10
G = 64
BN_EPS = 1e-5

NC = 2
NS = 16
NW = NC * NS
CHUNK = 125
NCHUNK = E // (NW * CHUNK)
N_PAD = 10240
ROWS_PER_TILE = N_PAD // NS
ZROWS = 32
NBUF = 5


def _segsum_body(h_hbm, ei_hbm, out_hbm,
                 sidx, didx, gbuf, zbuf, acc, *gsem):
    c = lax.axis_index("c")
    s = lax.axis_index("s")
    wid = c * NS + s

    for r in range(ZROWS):
        for q in range(H // 16):
            zbuf[r, pl.ds(q * 16, 16)] = jnp.zeros((16,), jnp.float32)
    base = s * ROWS_PER_TILE

    def zloop(k, carry):
        pltpu.sync_copy(zbuf, acc.at[pl.ds(base + k * ZROWS, ZROWS)])
        return carry

    lax.fori_loop(0, ROWS_PER_TILE // ZROWS, zloop, 0)

    pltpu.sync_copy(ei_hbm.at[0, wid], sidx)
    pltpu.sync_copy(ei_hbm.at[1, wid], didx)

    plsc.subcore_barrier()

    for b in range(NBUF):
        pltpu.async_copy(h_hbm.at[sidx.at[b]], gbuf.at[b], gsem[b])

    def group(io, carry):
        jo = io * NBUF
        for b in range(NBUF):
            j = jo + b
            pltpu.make_async_copy(h_hbm.at[sidx.at[b]],
                                  gbuf.at[b], gsem[b]).wait()
            pltpu.sync_copy(gbuf.at[b], acc.at[didx.at[j]], add=True)
            pltpu.async_copy(h_hbm.at[sidx.at[j + NBUF]], gbuf.at[b],
                             gsem[b])
        return carry

    lax.fori_loop(0, NCHUNK // NBUF - 1, group, 0)

    jo = NCHUNK - NBUF
    for b in range(NBUF):
        pltpu.make_async_copy(h_hbm.at[sidx.at[b]],
                              gbuf.at[b], gsem[b]).wait()
        pltpu.sync_copy(gbuf.at[b], acc.at[didx.at[jo + b]], add=True)

    plsc.subcore_barrier()
    pltpu.sync_copy(acc.at[pl.ds(base, ROWS_PER_TILE)],
                    out_hbm.at[c, pl.ds(base, ROWS_PER_TILE)])


_segsum_call = pl.kernel(
    _segsum_body,
    out_type=jax.ShapeDtypeStruct((NC, N_PAD, H), jnp.float32),
    mesh=plsc.VectorSubcoreMesh(core_axis_name="c", subcore_axis_name="s",
                                num_cores=NC, num_subcores=NS),
    scratch_types=[
        pltpu.VMEM((NCHUNK, CHUNK), jnp.int32),
        pltpu.VMEM((NCHUNK, CHUNK), jnp.int32),
        pltpu.VMEM((NBUF, CHUNK, H), jnp.float32),
        pltpu.VMEM((ZROWS, H), jnp.float32),
        pltpu.VMEM_SHARED((N_PAD, H), jnp.float32),
    ] + [pltpu.SemaphoreType.DMA] * NBUF,
    compiler_params=pltpu.CompilerParams(use_tc_tiling_on_sc=False,
                                         needs_layout_passes=False),
    name="gin_segsum_sc",
)


def _enc_body(x_ref, w_ref, b_ref, out_ref):
    out_ref[:, :] = jnp.dot(x_ref[:, :], w_ref[:, :],
                            preferred_element_type=jnp.float32) + b_ref[:, :]


_enc_call = pl.pallas_call(
    _enc_body,
    out_shape=jax.ShapeDtypeStruct((N, H), jnp.float32),
    name="gin_encoder_tc",
)


def _mlp_body(h_ref, parts_ref, w1_ref, b1_ref, w2_ref, b2_ref,
              gam_ref, bet_ref, eps_ref, out_ref):
    a = ((1.0 + eps_ref[0, 0]) * h_ref[:, :]
         + parts_ref[0, :N, :] + parts_ref[1, :N, :])
    t = jnp.dot(a, w1_ref[:, :], preferred_element_type=jnp.float32)
    t = jnp.maximum(t + b1_ref[:, :], 0.0)
    h2 = jnp.dot(t, w2_ref[:, :], preferred_element_type=jnp.float32)
    h2 = h2 + b2_ref[:, :]
    mean = jnp.mean(h2, axis=0, keepdims=True)
    var = jnp.mean((h2 - mean) ** 2, axis=0, keepdims=True)
    hn = (h2 - mean) / jnp.sqrt(var + BN_EPS) * gam_ref[:, :] + bet_ref[:, :]
    out_ref[:, :] = jnp.maximum(hn, 0.0)


_mlp_call = pl.pallas_call(
    _mlp_body,
    out_shape=jax.ShapeDtypeStruct((N, H), jnp.float32),
    name="gin_layer_tc",
)


def _pool_body(h_ref, batch_ref, w_ref, b_ref, out_ref):
    gids = lax.broadcasted_iota(jnp.int32, (G, 1), 0)
    mask = (batch_ref[:, :] == gids).astype(jnp.float32)
    sums = jnp.dot(mask, h_ref[:, :], preferred_element_type=jnp.float32)
    counts = jnp.sum(mask, axis=1, keepdims=True)
    pooled = sums / jnp.maximum(counts, 1.0)
    out_ref[:, :] = jnp.dot(pooled, w_ref[:, :],
                            preferred_element_type=jnp.float32) + b_ref[:, :]


_pool_call = pl.pallas_call(
    _pool_body,
    out_shape=jax.ShapeDtypeStruct((G, C), jnp.float32),
    name="gin_pool_tc",
)


def kernel(x, edge_index, batch, enc_W, enc_b, eps, W1, b1, W2, b2,
           gamma, beta, lin_W, lin_b):
    ei = edge_index.reshape(2, NW, NCHUNK, CHUNK)
    h = _enc_call(x, enc_W, enc_b.reshape(1, H))
    for i in range(L):
        parts = _segsum_call(h, ei)
        h = _mlp_call(h, parts, W1[i], b1[i].reshape(1, H),
                      W2[i], b2[i].reshape(1, H), gamma[i].reshape(1, H),
                      beta[i].reshape(1, H), eps[i].reshape(1, 1))
    return _pool_call(h, batch.reshape(1, N), lin_W, lin_b.reshape(1, C))


# speedup vs baseline: 1.8230x; 1.0711x over previous
import functools

import jax
import jax.numpy as jnp
from jax import lax
from jax.experimental import pallas as pl
from jax.experimental.pallas import tpu as pltpu
from jax.experimental.pallas import tpu_sc as plsc

N = 10000
E = 320000
F_IN = 128
H = 64
L = 3
C = 10
G = 64
BN_EPS = 1e-5

NC = 2
NS = 16
NW = NC * NS
CHUNK = 80
NCHUNK = E // (NW * CHUNK)
N_PAD = 10240
ROWS_PER_TILE = N_PAD // NS
ZROWS = 32
NBUF = 5


def _segsum_body(h_hbm, ei_hbm, out_hbm,
                 sidx, didx, gbuf, zbuf, acc, *gsem):
    c = lax.axis_index("c")
    s = lax.axis_index("s")
    wid = c * NS + s

    pltpu.async_copy(ei_hbm.at[0, wid], sidx, gsem[0])
    pltpu.async_copy(ei_hbm.at[1, wid], didx, gsem[1])

    for r in range(ZROWS):
        for q in range(H // 16):
            zbuf[r, pl.ds(q * 16, 16)] = jnp.zeros((16,), jnp.float32)
    base = s * ROWS_PER_TILE

    def zloop(k, carry):
        pltpu.sync_copy(zbuf, acc.at[pl.ds(base + k * ZROWS, ZROWS)])
        return carry

    lax.fori_loop(0, ROWS_PER_TILE // ZROWS, zloop, 0)

    pltpu.make_async_copy(ei_hbm.at[0, wid], sidx, gsem[0]).wait()
    pltpu.make_async_copy(ei_hbm.at[1, wid], didx, gsem[1]).wait()

    plsc.subcore_barrier()

    for b in range(NBUF):
        pltpu.async_copy(h_hbm.at[sidx.at[b]], gbuf.at[b], gsem[b])

    def group(io, carry):
        jo = io * NBUF
        for b in range(NBUF):
            j = jo + b
            pltpu.make_async_copy(h_hbm.at[sidx.at[b]],
                                  gbuf.at[b], gsem[b]).wait()
            pltpu.sync_copy(gbuf.at[b], acc.at[didx.at[j]], add=True)
            pltpu.async_copy(h_hbm.at[sidx.at[j + NBUF]], gbuf.at[b],
                             gsem[b])
        return carry

    lax.fori_loop(0, NCHUNK // NBUF - 1, group, 0)

    jo = NCHUNK - NBUF
    for b in range(NBUF):
        pltpu.make_async_copy(h_hbm.at[sidx.at[b]],
                              gbuf.at[b], gsem[b]).wait()
        pltpu.sync_copy(gbuf.at[b], acc.at[didx.at[jo + b]], add=True)

    plsc.subcore_barrier()
    pltpu.sync_copy(acc.at[pl.ds(base, ROWS_PER_TILE)],
                    out_hbm.at[c, pl.ds(base, ROWS_PER_TILE)])


_segsum_call = pl.kernel(
    _segsum_body,
    out_type=jax.ShapeDtypeStruct((NC, N_PAD, H), jnp.float32),
    mesh=plsc.VectorSubcoreMesh(core_axis_name="c", subcore_axis_name="s",
                                num_cores=NC, num_subcores=NS),
    scratch_types=[
        pltpu.VMEM((NCHUNK, CHUNK), jnp.int32),
        pltpu.VMEM((NCHUNK, CHUNK), jnp.int32),
        pltpu.VMEM((NBUF, CHUNK, H), jnp.float32),
        pltpu.VMEM((ZROWS, H), jnp.float32),
        pltpu.VMEM_SHARED((N_PAD, H), jnp.float32),
    ] + [pltpu.SemaphoreType.DMA] * NBUF,
    compiler_params=pltpu.CompilerParams(use_tc_tiling_on_sc=False,
                                         needs_layout_passes=False),
    name="gin_segsum_sc",
)


def _enc_body(x_ref, w_ref, b_ref, out_ref):
    out_ref[:, :] = jnp.dot(x_ref[:, :], w_ref[:, :],
                            preferred_element_type=jnp.float32) + b_ref[:, :]


_enc_call = pl.pallas_call(
    _enc_body,
    out_shape=jax.ShapeDtypeStruct((N, H), jnp.float32),
    name="gin_encoder_tc",
)


def _layer_math(h_ref, parts_ref, w1_ref, b1_ref, w2_ref, b2_ref,
                gam_ref, bet_ref, eps_ref):
    a = ((1.0 + eps_ref[0, 0]) * h_ref[:, :]
         + parts_ref[0, :N, :] + parts_ref[1, :N, :])
    t = jnp.dot(a, w1_ref[:, :], preferred_element_type=jnp.float32)
    t = jnp.maximum(t + b1_ref[:, :], 0.0)
    h2 = jnp.dot(t, w2_ref[:, :], preferred_element_type=jnp.float32)
    h2 = h2 + b2_ref[:, :]
    mean = jnp.mean(h2, axis=0, keepdims=True)
    var = jnp.mean((h2 - mean) ** 2, axis=0, keepdims=True)
    hn = (h2 - mean) / jnp.sqrt(var + BN_EPS) * gam_ref[:, :] + bet_ref[:, :]
    return jnp.maximum(hn, 0.0)


def _mlp_body(h_ref, parts_ref, w1_ref, b1_ref, w2_ref, b2_ref,
              gam_ref, bet_ref, eps_ref, out_ref):
    out_ref[:, :] = _layer_math(h_ref, parts_ref, w1_ref, b1_ref, w2_ref,
                                b2_ref, gam_ref, bet_ref, eps_ref)


_mlp_call = pl.pallas_call(
    _mlp_body,
    out_shape=jax.ShapeDtypeStruct((N, H), jnp.float32),
    name="gin_layer_tc",
)


def _mlp_pool_body(h_ref, parts_ref, w1_ref, b1_ref, w2_ref, b2_ref,
                   gam_ref, bet_ref, eps_ref, batch_ref, lw_ref, lb_ref,
                   out_ref):
    hfin = _layer_math(h_ref, parts_ref, w1_ref, b1_ref, w2_ref, b2_ref,
                       gam_ref, bet_ref, eps_ref)
    gids = lax.broadcasted_iota(jnp.int32, (G, 1), 0)
    mask = (batch_ref[:, :] == gids).astype(jnp.float32)
    sums = jnp.dot(mask, hfin, preferred_element_type=jnp.float32)
    counts = jnp.sum(mask, axis=1, keepdims=True)
    pooled = sums / jnp.maximum(counts, 1.0)
    out_ref[:, :] = jnp.dot(pooled, lw_ref[:, :],
                            preferred_element_type=jnp.float32) + lb_ref[:, :]


_mlp_pool_call = pl.pallas_call(
    _mlp_pool_body,
    out_shape=jax.ShapeDtypeStruct((G, C), jnp.float32),
    name="gin_layer_pool_tc",
)


def kernel(x, edge_index, batch, enc_W, enc_b, eps, W1, b1, W2, b2,
           gamma, beta, lin_W, lin_b):
    ei = edge_index.reshape(2, NW, NCHUNK, CHUNK)
    h = _enc_call(x, enc_W, enc_b.reshape(1, H))
    for i in range(L - 1):
        parts = _segsum_call(h, ei)
        h = _mlp_call(h, parts, W1[i], b1[i].reshape(1, H),
                      W2[i], b2[i].reshape(1, H), gamma[i].reshape(1, H),
                      beta[i].reshape(1, H), eps[i].reshape(1, 1))
    parts = _segsum_call(h, ei)
    i = L - 1
    return _mlp_pool_call(h, parts, W1[i], b1[i].reshape(1, H),
                          W2[i], b2[i].reshape(1, H), gamma[i].reshape(1, H),
                          beta[i].reshape(1, H), eps[i].reshape(1, 1),
                          batch.reshape(1, N), lin_W, lin_b.reshape(1, C))


# speedup vs baseline: 2.2586x; 1.2389x over previous
import jax
import jax.numpy as jnp
from jax import lax
from jax.experimental import pallas as pl
from jax.experimental.pallas import tpu as pltpu
from jax.experimental.pallas import tpu_sc as plsc

N = 10000
E = 320000
F_IN = 128
H = 64
L = 3
C = 10
G = 64
BN_EPS = 1e-5

NC = 2
NS = 16
NW = NC * NS
CHUNK = 80
NCHUNK = E // (NW * CHUNK)
N_PAD = 10240
ROWS_PER_TILE = N_PAD // NS
ZROWS = 32
NBUF = 5


def _segsum_body(h_hbm, ei_hbm, out_hbm,
                 sidx, didx, gbuf, zbuf, acc, *gsem):
    c = lax.axis_index("c")
    s = lax.axis_index("s")
    wid = c * NS + s

    pltpu.async_copy(ei_hbm.at[0, wid], sidx, gsem[0])
    pltpu.async_copy(ei_hbm.at[1, wid], didx, gsem[1])

    for r in range(ZROWS):
        for q in range(H // 16):
            zbuf[r, pl.ds(q * 16, 16)] = jnp.zeros((16,), jnp.float32)
    base = s * ROWS_PER_TILE

    def zloop(k, carry):
        pltpu.sync_copy(zbuf, acc.at[pl.ds(base + k * ZROWS, ZROWS)])
        return carry

    lax.fori_loop(0, ROWS_PER_TILE // ZROWS, zloop, 0)

    pltpu.make_async_copy(ei_hbm.at[0, wid], sidx, gsem[0]).wait()
    pltpu.make_async_copy(ei_hbm.at[1, wid], didx, gsem[1]).wait()

    plsc.subcore_barrier()

    for b in range(NBUF):
        pltpu.async_copy(h_hbm.at[sidx.at[b]], gbuf.at[b], gsem[b])

    def group(io, carry):
        jo = io * NBUF
        for b in range(NBUF):
            j = jo + b
            pltpu.make_async_copy(h_hbm.at[sidx.at[b]],
                                  gbuf.at[b], gsem[b]).wait()
            pltpu.sync_copy(gbuf.at[b], acc.at[didx.at[j]], add=True)
            pltpu.async_copy(h_hbm.at[sidx.at[j + NBUF]], gbuf.at[b],
                             gsem[b])
        return carry

    lax.fori_loop(0, NCHUNK // NBUF - 1, group, 0)

    jo = NCHUNK - NBUF
    for b in range(NBUF):
        pltpu.make_async_copy(h_hbm.at[sidx.at[b]],
                              gbuf.at[b], gsem[b]).wait()
        pltpu.sync_copy(gbuf.at[b], acc.at[didx.at[jo + b]], add=True)

    plsc.subcore_barrier()
    pltpu.sync_copy(acc.at[pl.ds(base, ROWS_PER_TILE)],
                    out_hbm.at[c, pl.ds(base, ROWS_PER_TILE)])


_segsum_call = pl.kernel(
    _segsum_body,
    out_type=jax.ShapeDtypeStruct((NC, N_PAD, H), jnp.float32),
    mesh=plsc.VectorSubcoreMesh(core_axis_name="c", subcore_axis_name="s",
                                num_cores=NC, num_subcores=NS),
    scratch_types=[
        pltpu.VMEM((NCHUNK, CHUNK), jnp.int32),
        pltpu.VMEM((NCHUNK, CHUNK), jnp.int32),
        pltpu.VMEM((NBUF, CHUNK, H), jnp.float32),
        pltpu.VMEM((ZROWS, H), jnp.float32),
        pltpu.VMEM_SHARED((N_PAD, H), jnp.float32),
    ] + [pltpu.SemaphoreType.DMA] * NBUF,
    compiler_params=pltpu.CompilerParams(use_tc_tiling_on_sc=False,
                                         needs_layout_passes=False),
    name="gin_segsum_sc",
)


def _enc_body(x_ref, w_ref, b_ref, out_ref):
    out_ref[:, :] = jnp.dot(x_ref[:, :], w_ref[:, :],
                            preferred_element_type=jnp.float32) + b_ref[:, :]


_enc_call = pl.pallas_call(
    _enc_body,
    out_shape=jax.ShapeDtypeStruct((N, H), jnp.float32),
    name="gin_encoder_tc",
)


N2 = N // 2


def _layer_math(h_ref, parts_ref, w1_ref, b1_ref, w2_ref, b2_ref,
                gam_ref, bet_ref, eps_ref):
    a = ((1.0 + eps_ref[0, 0]) * h_ref[:, :]
         + parts_ref[0, :N2, :] + parts_ref[1, :N2, :])
    t = jnp.dot(a, w1_ref[:, :], preferred_element_type=jnp.float32)
    t = jnp.maximum(t + b1_ref[:, :], 0.0)
    h2 = jnp.dot(t, w2_ref[:, :], preferred_element_type=jnp.float32)
    h2 = h2 + b2_ref[:, :]
    m = jnp.mean(h2, axis=0, keepdims=True)
    s2 = jnp.mean(h2 * h2, axis=0, keepdims=True)
    mean = 0.5 * (m[:, :H] + m[:, H:])
    var = 0.5 * (s2[:, :H] + s2[:, H:]) - mean * mean
    mean2 = jnp.concatenate([mean, mean], axis=1)
    var2 = jnp.concatenate([var, var], axis=1)
    hn = (h2 - mean2) / jnp.sqrt(var2 + BN_EPS) * gam_ref[:, :] + bet_ref[:, :]
    return jnp.maximum(hn, 0.0)


def _mlp_body(h_ref, parts_ref, w1_ref, b1_ref, w2_ref, b2_ref,
              gam_ref, bet_ref, eps_ref, out_ref):
    out_ref[:, :] = _layer_math(h_ref, parts_ref, w1_ref, b1_ref, w2_ref,
                                b2_ref, gam_ref, bet_ref, eps_ref)


_mlp_call = pl.pallas_call(
    _mlp_body,
    out_shape=jax.ShapeDtypeStruct((N2, 2 * H), jnp.float32),
    name="gin_layer_tc",
)


def _mlp_pool_body(h_ref, parts_ref, w1_ref, b1_ref, w2_ref, b2_ref,
                   gam_ref, bet_ref, eps_ref, be_ref, bo_ref, lw_ref,
                   lb_ref, out_ref):
    hfin = _layer_math(h_ref, parts_ref, w1_ref, b1_ref, w2_ref, b2_ref,
                       gam_ref, bet_ref, eps_ref)
    gids = lax.broadcasted_iota(jnp.int32, (G, 1), 0)
    mask_e = (be_ref[:, :] == gids).astype(jnp.float32)
    mask_o = (bo_ref[:, :] == gids).astype(jnp.float32)
    sums_e = jnp.dot(mask_e, hfin, preferred_element_type=jnp.float32)
    sums_o = jnp.dot(mask_o, hfin, preferred_element_type=jnp.float32)
    sums = sums_e[:, :H] + sums_o[:, H:]
    counts = (jnp.sum(mask_e, axis=1, keepdims=True)
              + jnp.sum(mask_o, axis=1, keepdims=True))
    pooled = sums / jnp.maximum(counts, 1.0)
    out_ref[:, :] = jnp.dot(pooled, lw_ref[:, :],
                            preferred_element_type=jnp.float32) + lb_ref[:, :]


_mlp_pool_call = pl.pallas_call(
    _mlp_pool_body,
    out_shape=jax.ShapeDtypeStruct((G, C), jnp.float32),
    name="gin_layer_pool_tc",
)


def _blockdiag(w):
    return jnp.kron(jnp.eye(2, dtype=w.dtype), w)


def _dup(v):
    return jnp.concatenate([v, v]).reshape(1, 2 * H)


def kernel(x, edge_index, batch, enc_W, enc_b, eps, W1, b1, W2, b2,
           gamma, beta, lin_W, lin_b):
    ei = edge_index.reshape(2, NW, NCHUNK, CHUNK)
    h = _enc_call(x, enc_W, enc_b.reshape(1, H)).reshape(N2, 2 * H)
    for i in range(L):
        parts = _segsum_call(h.reshape(N, H), ei)
        parts = parts.reshape(2, N_PAD // 2, 2 * H)
        args = (h, parts, _blockdiag(W1[i]), _dup(b1[i]),
                _blockdiag(W2[i]), _dup(b2[i]), _dup(gamma[i]),
                _dup(beta[i]), eps[i].reshape(1, 1))
        if i < L - 1:
            h = _mlp_call(*args)
        else:
            return _mlp_pool_call(*args, batch[0::2].reshape(1, N2),
                                  batch[1::2].reshape(1, N2), lin_W,
                                  lin_b.reshape(1, C))
